# Initial kernel scaffold; baseline (speedup 1.0000x reference)
import functools

import jax
import jax.numpy as jnp
from jax import lax
from jax.experimental import pallas as pl
from jax.experimental.pallas import tpu as pltpu
from jax.experimental.pallas import tpu_sc as plsc

N_NODES = 50000
HALF = 16
LAYERS = 3
NNZ = 800000
BATCH = 4096

NC = 2
NS = 16
LANES = 16

N_PAD = 50048
ROWS_PER_TILE = N_PAD // NS
CHUNK = 2048
SUB = CHUNK // 128
CHUNKS_PER_TILE = 25
NNZ_PAD = NS * CHUNKS_PER_TILE * CHUNK
B_PER_TILE = BATCH // NS
BSUB = B_PER_TILE // 128
MUL_UNROLL = 8


def _gcn_body(ue_ref, ie_ref, val_ref, ueo_ref, ieo_ref, u0_ref, i0_ref,
              busr_ref, bitm_ref,
              scores_ref, u1_ref, i1_ref, u2_ref, i2_ref, u3_ref, i3_ref,
              uacc, iacc, zbuf, ue_raw, ie_raw, ue_off, ie_off, val_f,
              u_rows, i_rows, bu_idx, bi_idx, fu, fi, sc_v, gsem, ssem):
    c = lax.axis_index("c")
    s = lax.axis_index("s")
    row0 = s * ROWS_PER_TILE
    tab_off = c * N_PAD

    zeros16 = jnp.zeros((LANES,), jnp.float32)

    def _zb(r, _):
        zbuf[r] = zeros16
        return 0
    lax.fori_loop(0, 512, _zb, 0)

    u_tabs = [u0_ref, u1_ref, u2_ref, u3_ref]
    i_tabs = [i0_ref, i1_ref, i2_ref, i3_ref]

    def layer_pass(src_u, src_i, dst_u, dst_i):
        for k in range(6):
            pltpu.sync_copy(zbuf, uacc.at[pl.ds(row0 + k * 512, 512)])
            pltpu.sync_copy(zbuf, iacc.at[pl.ds(row0 + k * 512, 512)])
        pltpu.sync_copy(zbuf.at[pl.ds(0, 56)],
                        uacc.at[pl.ds(row0 + 3072, 56)])
        pltpu.sync_copy(zbuf.at[pl.ds(0, 56)],
                        iacc.at[pl.ds(row0 + 3072, 56)])
        plsc.subcore_barrier()

        def chunk_body(t, _):
            ebase = s * (CHUNKS_PER_TILE * SUB) + t * SUB
            pltpu.sync_copy(ue_ref.at[pl.ds(ebase, SUB)], ue_raw)
            pltpu.sync_copy(ie_ref.at[pl.ds(ebase, SUB)], ie_raw)
            pltpu.sync_copy(ueo_ref.at[c, pl.ds(ebase, SUB)], ue_off)
            pltpu.sync_copy(ieo_ref.at[c, pl.ds(ebase, SUB)], ie_off)
            pltpu.sync_copy(val_ref.at[pl.ds(ebase * 128, CHUNK)], val_f)
            descs = []
            for j in range(SUB):
                descs.append(pltpu.async_copy(
                    src_u.at[ue_off.at[j]],
                    u_rows.at[pl.ds(j * 128, 128)], gsem))
                descs.append(pltpu.async_copy(
                    src_i.at[ie_off.at[j]],
                    i_rows.at[pl.ds(j * 128, 128)], gsem))
            for d in descs:
                d.wait()

            def mul_body(r, _):
                for q in range(MUL_UNROLL):
                    rr = r * MUL_UNROLL + q
                    v = val_f[rr]
                    u_rows[rr] = u_rows[rr] * v
                    i_rows[rr] = i_rows[rr] * v
                return 0
            lax.fori_loop(0, CHUNK // MUL_UNROLL, mul_body, 0)

            descs = []
            for j in range(SUB):
                descs.append(pltpu.async_copy(
                    u_rows.at[pl.ds(j * 128, 128)],
                    iacc.at[ie_raw.at[j]], ssem, add=True))
                descs.append(pltpu.async_copy(
                    i_rows.at[pl.ds(j * 128, 128)],
                    uacc.at[ue_raw.at[j]], ssem, add=True))
            for d in descs:
                d.wait()
            return 0
        lax.fori_loop(0, CHUNKS_PER_TILE, chunk_body, 0)
        plsc.subcore_barrier()
        pltpu.sync_copy(uacc.at[pl.ds(row0, ROWS_PER_TILE)],
                        dst_u.at[pl.ds(tab_off + row0, ROWS_PER_TILE)])
        pltpu.sync_copy(iacc.at[pl.ds(row0, ROWS_PER_TILE)],
                        dst_i.at[pl.ds(tab_off + row0, ROWS_PER_TILE)])
        plsc.subcore_barrier()

    layer_pass(u_tabs[0], i_tabs[0], u1_ref, i1_ref)
    layer_pass(u_tabs[1], i_tabs[1], u2_ref, i2_ref)
    layer_pass(u_tabs[2], i_tabs[2], u3_ref, i3_ref)

    pltpu.sync_copy(busr_ref.at[c, pl.ds(s * BSUB, BSUB)], bu_idx)
    pltpu.sync_copy(bitm_ref.at[c, pl.ds(s * BSUB, BSUB)], bi_idx)

    def _zf(r, _):
        fu[r] = zeros16
        fi[r] = zeros16
        return 0
    lax.fori_loop(0, B_PER_TILE, _zf, 0)

    descs = []
    for l in range(LAYERS + 1):
        for j in range(BSUB):
            descs.append(pltpu.async_copy(
                u_tabs[l].at[bu_idx.at[j]],
                fu.at[pl.ds(j * 128, 128)], gsem, add=True))
            descs.append(pltpu.async_copy(
                i_tabs[l].at[bi_idx.at[j]],
                fi.at[pl.ds(j * 128, 128)], gsem, add=True))
    for d in descs:
        d.wait()

    def dot_body(r, _):
        p = fu[r] * fi[r]
        sc_v[r] = jnp.sum(p) * (1.0 / 16.0)
        return 0
    lax.fori_loop(0, B_PER_TILE, dot_body, 0)
    pltpu.sync_copy(sc_v, scores_ref.at[c, pl.ds(s * B_PER_TILE, B_PER_TILE)])


_TAB = jax.ShapeDtypeStruct((NC * N_PAD, HALF), jnp.float32)

_gcn_kernel = functools.partial(
    pl.kernel,
    out_type=(jax.ShapeDtypeStruct((NC, BATCH), jnp.float32),
              _TAB, _TAB, _TAB, _TAB, _TAB, _TAB),
    mesh=plsc.VectorSubcoreMesh(core_axis_name="c", subcore_axis_name="s",
                                num_cores=NC, num_subcores=NS),
    scratch_types=(
        pltpu.VMEM_SHARED((N_PAD, HALF), jnp.float32),
        pltpu.VMEM_SHARED((N_PAD, HALF), jnp.float32),
        pltpu.VMEM((512, HALF), jnp.float32),
        pltpu.VMEM((SUB, 128), jnp.int32),
        pltpu.VMEM((SUB, 128), jnp.int32),
        pltpu.VMEM((SUB, 128), jnp.int32),
        pltpu.VMEM((SUB, 128), jnp.int32),
        pltpu.VMEM((CHUNK,), jnp.float32),
        pltpu.VMEM((CHUNK, HALF), jnp.float32),
        pltpu.VMEM((CHUNK, HALF), jnp.float32),
        pltpu.VMEM((BSUB, 128), jnp.int32),
        pltpu.VMEM((BSUB, 128), jnp.int32),
        pltpu.VMEM((B_PER_TILE, HALF), jnp.float32),
        pltpu.VMEM((B_PER_TILE, HALF), jnp.float32),
        pltpu.VMEM((B_PER_TILE,), jnp.float32),
        pltpu.SemaphoreType.DMA,
        pltpu.SemaphoreType.DMA,
    ),
)(_gcn_body)


def kernel(users, items, edge_index, edge_vals, user_table, item_table):
    edge_u = edge_index[0]
    edge_i = edge_index[1]
    pad = NNZ_PAD - NNZ
    ue = jnp.pad(edge_u, (0, pad)).reshape(NNZ_PAD // 128, 128)
    ie = jnp.pad(edge_i, (0, pad)).reshape(NNZ_PAD // 128, 128)
    vals = jnp.pad(edge_vals, (0, pad))
    ueo = jnp.stack([ue, ue + N_PAD], axis=0)
    ieo = jnp.stack([ie, ie + N_PAD], axis=0)
    ut = jnp.pad(user_table, ((0, N_PAD - N_NODES), (0, 0)))
    it = jnp.pad(item_table, ((0, N_PAD - N_NODES), (0, 0)))
    u0 = jnp.concatenate([ut[:, :HALF], ut[:, HALF:]], axis=0)
    i0 = jnp.concatenate([it[:, :HALF], it[:, HALF:]], axis=0)
    bu = users.reshape(BATCH // 128, 128)
    bi = items.reshape(BATCH // 128, 128)
    busr = jnp.stack([bu, bu + N_PAD], axis=0)
    bitm = jnp.stack([bi, bi + N_PAD], axis=0)

    outs = _gcn_kernel(ue, ie, vals, ueo, ieo, u0, i0, busr, bitm)
    part = outs[0]
    return part[0] + part[1]


# speedup vs baseline: 9.3718x; 9.3718x over previous
import functools

import jax
import jax.numpy as jnp
from jax import lax
from jax.experimental import pallas as pl
from jax.experimental.pallas import tpu as pltpu
from jax.experimental.pallas import tpu_sc as plsc

N_NODES = 50000
HALF = 16
LAYERS = 3
NNZ = 800000
BATCH = 4096

NC = 2
NS = 16
LANES = 16

N_PAD = 50048
ROWS_PER_TILE = N_PAD // NS
CHUNK = 512
SUB = CHUNK // 128
CHUNKS_PER_TILE = 98
NNZ_PAD = NS * CHUNKS_PER_TILE * CHUNK
B_PER_TILE = BATCH // NS


def _gcn_body(ue_ref, ie_ref, val_ref, ueo_ref, ieo_ref, u0_ref, i0_ref,
              busr_ref, bitm_ref,
              scores_ref, u1_ref, i1_ref, u2_ref, i2_ref, u3_ref, i3_ref,
              uacc, iacc, ue_raw, ie_raw, ue_off, ie_off, val_f,
              u_rows, i_rows, bu_idx, bi_idx, fu, fi, sc_v, gsem, ssem):
    c = lax.axis_index("c")
    s = lax.axis_index("s")
    row0 = s * ROWS_PER_TILE
    tab_off = c * N_PAD

    zeros16 = jnp.zeros((LANES,), jnp.float32)

    u_tabs = [u0_ref, u1_ref, u2_ref, u3_ref]
    i_tabs = [i0_ref, i1_ref, i2_ref, i3_ref]

    def layer_pass(src_u, src_i, dst_u, dst_i):
        def _zb(r, _):
            u_rows[r] = zeros16
            i_rows[r] = zeros16
            return 0
        lax.fori_loop(0, CHUNK, _zb, 0)
        for k in range(6):
            pltpu.sync_copy(u_rows, uacc.at[pl.ds(row0 + k * 512, 512)])
            pltpu.sync_copy(i_rows, iacc.at[pl.ds(row0 + k * 512, 512)])
        pltpu.sync_copy(u_rows.at[pl.ds(0, 56)],
                        uacc.at[pl.ds(row0 + 3072, 56)])
        pltpu.sync_copy(i_rows.at[pl.ds(0, 56)],
                        iacc.at[pl.ds(row0 + 3072, 56)])
        plsc.subcore_barrier()

        def chunk_body(t, _):
            ebase = s * (CHUNKS_PER_TILE * SUB) + t * SUB
            pltpu.sync_copy(ue_ref.at[pl.ds(ebase, SUB)], ue_raw)
            pltpu.sync_copy(ie_ref.at[pl.ds(ebase, SUB)], ie_raw)
            pltpu.sync_copy(ueo_ref.at[c, pl.ds(ebase, SUB)], ue_off)
            pltpu.sync_copy(ieo_ref.at[c, pl.ds(ebase, SUB)], ie_off)
            pltpu.sync_copy(val_ref.at[pl.ds(ebase * 128, CHUNK)], val_f)
            descs = []
            for j in range(SUB):
                descs.append(pltpu.async_copy(
                    src_u.at[ue_off.at[j]],
                    u_rows.at[pl.ds(j * 128, 128)], gsem))
                descs.append(pltpu.async_copy(
                    src_i.at[ie_off.at[j]],
                    i_rows.at[pl.ds(j * 128, 128)], gsem))
            for d in descs:
                d.wait()

            def mul_body(g, _):
                vv = val_f[pl.ds(g * LANES, LANES)]
                for q in range(LANES):
                    rr = g * LANES + q
                    v = vv[q]
                    u_rows[rr] = u_rows[rr] * v
                    i_rows[rr] = i_rows[rr] * v
                return 0
            lax.fori_loop(0, CHUNK // LANES, mul_body, 0)

            descs = []
            for j in range(SUB):
                descs.append(pltpu.async_copy(
                    u_rows.at[pl.ds(j * 128, 128)],
                    iacc.at[ie_raw.at[j]], ssem, add=True))
                descs.append(pltpu.async_copy(
                    i_rows.at[pl.ds(j * 128, 128)],
                    uacc.at[ue_raw.at[j]], ssem, add=True))
            for d in descs:
                d.wait()
            return 0
        lax.fori_loop(0, CHUNKS_PER_TILE, chunk_body, 0)
        plsc.subcore_barrier()
        pltpu.sync_copy(uacc.at[pl.ds(row0, ROWS_PER_TILE)],
                        dst_u.at[pl.ds(tab_off + row0, ROWS_PER_TILE)])
        pltpu.sync_copy(iacc.at[pl.ds(row0, ROWS_PER_TILE)],
                        dst_i.at[pl.ds(tab_off + row0, ROWS_PER_TILE)])
        plsc.subcore_barrier()

    layer_pass(u_tabs[0], i_tabs[0], u1_ref, i1_ref)
    layer_pass(u_tabs[1], i_tabs[1], u2_ref, i2_ref)
    layer_pass(u_tabs[2], i_tabs[2], u3_ref, i3_ref)

    iota16 = lax.iota(jnp.int32, LANES)
    dnums = lax.GatherDimensionNumbers(
        offset_dims=(), collapsed_slice_dims=(0,), start_index_map=(0,))

    def _take16(v, idx):
        return lax.gather(v, idx[:, None], dimension_numbers=dnums,
                          slice_sizes=(1,),
                          mode=lax.GatherScatterMode.PROMISE_IN_BOUNDS)

    perms = [iota16 ^ m for m in (1, 2, 4, 8)]

    def _lane_sum(p):
        for m in perms:
            p = p + _take16(p, m)
        return p

    for p_half in range(2):
        pltpu.sync_copy(busr_ref.at[c, pl.ds(s * 2 + p_half, 1)], bu_idx)
        pltpu.sync_copy(bitm_ref.at[c, pl.ds(s * 2 + p_half, 1)], bi_idx)

        def _zf(r, _):
            fu[r] = zeros16
            fi[r] = zeros16
            return 0
        lax.fori_loop(0, 128, _zf, 0)

        descs = []
        for l in range(LAYERS + 1):
            descs.append(pltpu.async_copy(
                u_tabs[l].at[bu_idx.at[0]], fu, gsem, add=True))
            descs.append(pltpu.async_copy(
                i_tabs[l].at[bi_idx.at[0]], fi, gsem, add=True))
        for d in descs:
            d.wait()

        def dot_body(g, _):
            acc = zeros16
            for q in range(LANES):
                r = g * LANES + q
                sval = _lane_sum(fu[r] * fi[r]) * (1.0 / 16.0)
                acc = jnp.where(iota16 == q, sval, acc)
            sc_v[pl.ds(g * LANES, LANES)] = acc
            return 0
        lax.fori_loop(0, 128 // LANES, dot_body, 0)
        pltpu.sync_copy(
            sc_v,
            scores_ref.at[c, pl.ds(s * B_PER_TILE + p_half * 128, 128)])


_TAB = jax.ShapeDtypeStruct((NC * N_PAD, HALF), jnp.float32)

_gcn_kernel = functools.partial(
    pl.kernel,
    out_type=(jax.ShapeDtypeStruct((NC, BATCH), jnp.float32),
              _TAB, _TAB, _TAB, _TAB, _TAB, _TAB),
    mesh=plsc.VectorSubcoreMesh(core_axis_name="c", subcore_axis_name="s",
                                num_cores=NC, num_subcores=NS),
    compiler_params=pltpu.CompilerParams(use_tc_tiling_on_sc=False),
    scratch_types=(
        pltpu.VMEM_SHARED((N_PAD, HALF), jnp.float32),
        pltpu.VMEM_SHARED((N_PAD, HALF), jnp.float32),
        pltpu.VMEM((SUB, 128), jnp.int32),
        pltpu.VMEM((SUB, 128), jnp.int32),
        pltpu.VMEM((SUB, 128), jnp.int32),
        pltpu.VMEM((SUB, 128), jnp.int32),
        pltpu.VMEM((CHUNK,), jnp.float32),
        pltpu.VMEM((CHUNK, HALF), jnp.float32),
        pltpu.VMEM((CHUNK, HALF), jnp.float32),
        pltpu.VMEM((1, 128), jnp.int32),
        pltpu.VMEM((1, 128), jnp.int32),
        pltpu.VMEM((128, HALF), jnp.float32),
        pltpu.VMEM((128, HALF), jnp.float32),
        pltpu.VMEM((128,), jnp.float32),
        pltpu.SemaphoreType.DMA,
        pltpu.SemaphoreType.DMA,
    ),
)(_gcn_body)


def kernel(users, items, edge_index, edge_vals, user_table, item_table):
    edge_u = edge_index[0]
    edge_i = edge_index[1]
    pad = NNZ_PAD - NNZ
    ue = jnp.pad(edge_u, (0, pad)).reshape(NNZ_PAD // 128, 128)
    ie = jnp.pad(edge_i, (0, pad)).reshape(NNZ_PAD // 128, 128)
    vals = jnp.pad(edge_vals, (0, pad))
    ueo = jnp.stack([ue, ue + N_PAD], axis=0)
    ieo = jnp.stack([ie, ie + N_PAD], axis=0)
    ut = jnp.pad(user_table, ((0, N_PAD - N_NODES), (0, 0)))
    it = jnp.pad(item_table, ((0, N_PAD - N_NODES), (0, 0)))
    u0 = jnp.concatenate([ut[:, :HALF], ut[:, HALF:]], axis=0)
    i0 = jnp.concatenate([it[:, :HALF], it[:, HALF:]], axis=0)
    bu = users.reshape(BATCH // 128, 128)
    bi = items.reshape(BATCH // 128, 128)
    busr = jnp.stack([bu, bu + N_PAD], axis=0)
    bitm = jnp.stack([bi, bi + N_PAD], axis=0)

    outs = _gcn_kernel(ue, ie, vals, ueo, ieo, u0, i0, busr, bitm)
    part = outs[0]
    return part[0] + part[1]
